# Initial kernel scaffold; baseline (speedup 1.0000x reference)
import jax
import jax.numpy as jnp
from jax.experimental import pallas as pl


def kernel(coords, features, W1, b1, W2, b2):
    raise NotImplementedError("write your pallas kernel here")


# speedup vs baseline: 11.9730x; 11.9730x over previous
import functools
import math

import jax
import jax.numpy as jnp
import numpy as np
from jax import lax
from jax.experimental import pallas as pl
from jax.experimental.pallas import tpu as pltpu
from jax.experimental.pallas import tpu_sc as plsc

_SAMPLE_RATIO = 0.25
_K = 32


def _q_kernel(coords_ref, feats_ref, w1c_ref, w1f_ref, b1_ref, q_ref):
    c = coords_ref[...]
    f = feats_ref[...]
    q = lax.dot(f, w1f_ref[...], precision=jax.lax.Precision.HIGHEST)
    q += lax.dot(c, w1c_ref[...], precision=jax.lax.Precision.HIGHEST)
    q_ref[...] = q + b1_ref[...]


def _compute_q(coords2d, feats2d, W1c, W1f, b1, blk=2048):
    BN = coords2d.shape[0]
    C_out = W1f.shape[1]
    return pl.pallas_call(
        _q_kernel,
        grid=(BN // blk,),
        in_specs=[
            pl.BlockSpec((blk, 8), lambda i: (i, 0)),
            pl.BlockSpec((blk, feats2d.shape[1]), lambda i: (i, 0)),
            pl.BlockSpec((8, C_out), lambda i: (0, 0)),
            pl.BlockSpec((feats2d.shape[1], C_out), lambda i: (0, 0)),
            pl.BlockSpec((1, C_out), lambda i: (0, 0)),
        ],
        out_specs=pl.BlockSpec((blk, C_out), lambda i: (i, 0)),
        out_shape=jax.ShapeDtypeStruct((BN, C_out), jnp.float32),
    )(coords2d, feats2d, W1c, W1f, b1)


def _topk_kernel(samp_ref, coords_ref, knn_ref, *, n, k, s_blk):
    b = pl.program_id(0)
    s = samp_ref[...]
    ct = coords_ref[0]

    cross = None
    for d in range(3):
        sh = s[:, d:d + 1].astype(jnp.bfloat16).astype(jnp.float32)
        xh = ct[d:d + 1, :].astype(jnp.bfloat16).astype(jnp.float32)
        t = sh * xh
        cross = t if cross is None else cross + t
    sn = jnp.sum(s * s, axis=1, keepdims=True)
    cn = jnp.sum(ct * ct, axis=0)[None, :]
    d2 = sn + (cn - 2.0 * cross)
    d2 = jnp.maximum(d2, 0.0)
    iota = lax.broadcasted_iota(jnp.int32, (s_blk, n), 1)
    key = (d2.view(jnp.int32) & jnp.int32(~4095)) | iota
    base = b * n
    big = jnp.int32(2**31 - 1)
    for j in range(k):
        m = jnp.min(key, axis=1, keepdims=True)
        knn_ref[0, j, :] = (m[:, 0] & 4095) + base
        key = jnp.where(key == m, big, key)


def _compute_knn(sampled2d, coords_t, B, S, N, s_blk=256):
    kern = functools.partial(_topk_kernel, n=N, k=_K, s_blk=s_blk)
    return pl.pallas_call(
        kern,
        grid=(B, S // s_blk),
        in_specs=[
            pl.BlockSpec((s_blk, 8), lambda b, i: (b * (S // s_blk) + i, 0)),
            pl.BlockSpec((1, 8, N), lambda b, i: (b, 0, 0)),
        ],
        out_specs=pl.BlockSpec((1, _K, s_blk), lambda b, i: (b, 0, i)),
        out_shape=jax.ShapeDtypeStruct((B, _K, S), jnp.int32),
    )(sampled2d, coords_t)


def _make_sc_gather(n_rows_out, D, chunk=128):
    info = plsc.get_sparse_core_info()
    num_cores = info.num_cores
    nw = num_cores * info.num_subcores
    per_w = n_rows_out // nw
    nchunk = per_w // chunk
    mesh = plsc.VectorSubcoreMesh(core_axis_name="c", subcore_axis_name="s")

    @functools.partial(
        pl.kernel,
        mesh=mesh,
        out_type=jax.ShapeDtypeStruct((n_rows_out, D), jnp.float32),
        scratch_types=[
            pltpu.VMEM((per_w,), jnp.int32),
            pltpu.VMEM((chunk, D), jnp.float32),
            pltpu.VMEM((chunk, D), jnp.float32),
            pltpu.SemaphoreType.DMA,
            pltpu.SemaphoreType.DMA,
        ],
    )
    def gather(idx_hbm, table_hbm, out_hbm, idx_v, buf0, buf1, sem0, sem1):
        wid = lax.axis_index("s") * num_cores + lax.axis_index("c")
        base = wid * per_w
        pltpu.sync_copy(idx_hbm.at[pl.ds(base, per_w)], idx_v)

        def body(i, _):
            c0 = i * 2 * chunk
            cp0 = pltpu.async_copy(
                table_hbm.at[idx_v.at[pl.ds(c0, chunk)]], buf0, sem0)
            cp1 = pltpu.async_copy(
                table_hbm.at[idx_v.at[pl.ds(c0 + chunk, chunk)]], buf1, sem1)
            cp0.wait()
            pltpu.sync_copy(buf0, out_hbm.at[pl.ds(base + c0, chunk)])
            cp1.wait()
            pltpu.sync_copy(buf1, out_hbm.at[pl.ds(base + c0 + chunk, chunk)])
            return ()

        lax.fori_loop(0, nchunk // 2, body, (), unroll=False)

    return gather


def _gelu_exact(x):
    return 0.5 * x * (1.0 + lax.erf(x * np.float32(1.0 / np.sqrt(2.0))))


def _mlp_kernel(g_ref, samp_ref, w1c_ref, w2_ref, b2_ref, out_ref, *, k, s_blk, c_out):
    p = lax.dot(samp_ref[...], w1c_ref[...],
                precision=jax.lax.Precision.HIGHEST)
    h1 = _gelu_exact(g_ref[...] - p[None, :, :]).reshape(k * s_blk, c_out)
    h2 = _gelu_exact(
        lax.dot(h1, w2_ref[...], precision=jax.lax.Precision.HIGHEST)
        + b2_ref[...])
    acc = h2[0:s_blk]
    for j in range(1, k):
        acc = jnp.maximum(acc, h2[j * s_blk:(j + 1) * s_blk])
    out_ref[...] = acc


def _compute_mlp(G, sampled2d, W1c, W2, b2, BS, C_out, s_blk=128):
    kern = functools.partial(_mlp_kernel, k=_K, s_blk=s_blk, c_out=C_out)
    return pl.pallas_call(
        kern,
        grid=(BS // s_blk,),
        in_specs=[
            pl.BlockSpec((_K, s_blk, C_out), lambda i: (0, i, 0)),
            pl.BlockSpec((s_blk, 8), lambda i: (i, 0)),
            pl.BlockSpec((8, C_out), lambda i: (0, 0)),
            pl.BlockSpec((C_out, C_out), lambda i: (0, 0)),
            pl.BlockSpec((1, C_out), lambda i: (0, 0)),
        ],
        out_specs=pl.BlockSpec((s_blk, C_out), lambda i: (i, 0)),
        out_shape=jax.ShapeDtypeStruct((BS, C_out), jnp.float32),
    )(G, sampled2d, W1c, W2, b2)


def kernel(coords, features, W1, b1, W2, b2):
    B, N, _ = coords.shape
    C_in = features.shape[-1]
    C_out = W2.shape[-1]
    S = max(1, int(math.ceil(N * _SAMPLE_RATIO)))
    if S == N:
        idx = jnp.arange(N, dtype=jnp.int32)
    else:
        idx = jnp.round(jnp.linspace(0.0, N - 1, S)).astype(jnp.int32)

    sampled = coords[:, idx, :]

    pad = jnp.zeros((B, N, 5), jnp.float32)
    coords_p = jnp.concatenate([coords, pad], axis=-1)
    sampled_p = coords_p[:, idx, :]
    W1c = jnp.concatenate([W1[:3], jnp.zeros((5, C_out), jnp.float32)], axis=0)
    W1f = W1[3:]

    Q = _compute_q(coords_p.reshape(B * N, 8),
                   features.reshape(B * N, C_in), W1c, W1f,
                   b1.reshape(1, C_out))

    knn = _compute_knn(sampled_p.reshape(B * S, 8),
                       jnp.transpose(coords_p, (0, 2, 1)), B, S, N)
    gidx = jnp.transpose(knn, (1, 0, 2)).reshape(_K * B * S)

    G = _make_sc_gather(_K * B * S, C_out)(gidx, Q)
    G = G.reshape(_K, B * S, C_out)

    pooled = _compute_mlp(G, sampled_p.reshape(B * S, 8), W1c, W2,
                          b2.reshape(1, C_out), B * S, C_out)
    return sampled, pooled.reshape(B, S, C_out)


# speedup vs baseline: 19.2402x; 1.6070x over previous
import functools
import math

import jax
import jax.numpy as jnp
import numpy as np
from jax import lax
from jax.experimental import pallas as pl
from jax.experimental.pallas import tpu as pltpu
from jax.experimental.pallas import tpu_sc as plsc

_SAMPLE_RATIO = 0.25
_K = 32


def _q_kernel(coords_ref, feats_ref, w1c_ref, w1f_ref, b1_ref, q_ref):
    c = coords_ref[...]
    f = feats_ref[...]
    q = lax.dot(f, w1f_ref[...], precision=jax.lax.Precision.DEFAULT)
    q += lax.dot(c, w1c_ref[...], precision=jax.lax.Precision.DEFAULT)
    q_ref[...] = q + b1_ref[...]


def _compute_q(coords2d, feats2d, W1c, W1f, b1, blk=2048):
    BN = coords2d.shape[0]
    C_out = W1f.shape[1]
    return pl.pallas_call(
        _q_kernel,
        grid=(BN // blk,),
        in_specs=[
            pl.BlockSpec((blk, 8), lambda i: (i, 0)),
            pl.BlockSpec((blk, feats2d.shape[1]), lambda i: (i, 0)),
            pl.BlockSpec((8, C_out), lambda i: (0, 0)),
            pl.BlockSpec((feats2d.shape[1], C_out), lambda i: (0, 0)),
            pl.BlockSpec((1, C_out), lambda i: (0, 0)),
        ],
        out_specs=pl.BlockSpec((blk, C_out), lambda i: (i, 0)),
        out_shape=jax.ShapeDtypeStruct((BN, C_out), jnp.float32),
    )(coords2d, feats2d, W1c, W1f, b1)


def _topk_kernel(samp_ref, coords_ref, knn_ref, *, n, k, s_blk):
    b = pl.program_id(0)
    s = samp_ref[...]
    ct = coords_ref[0]

    cross = None
    for d in range(3):
        sh = s[:, d:d + 1].astype(jnp.bfloat16).astype(jnp.float32)
        xh = ct[d:d + 1, :].astype(jnp.bfloat16).astype(jnp.float32)
        t = sh * xh
        cross = t if cross is None else cross + t
    sn = jnp.sum(s * s, axis=1, keepdims=True)
    cn = jnp.sum(ct * ct, axis=0)[None, :]
    d2 = sn + (cn - 2.0 * cross)
    d2 = jnp.maximum(d2, 0.0)
    iota = lax.broadcasted_iota(jnp.int32, (s_blk, n), 1)
    key = (d2.view(jnp.int32) & jnp.int32(~4095)) | iota
    base = b * n
    big = jnp.int32(2**31 - 1)

    R = 6
    nch = n // 128
    chunks = [key[:, c * 128:(c + 1) * 128] for c in range(nch)]
    cand = []
    for r in range(R):
        f = chunks[0]
        for c in range(1, nch):
            f = jnp.minimum(f, chunks[c])
        cand.append(f)
        if r < R - 1:
            chunks = [jnp.where(ch == f, big, ch) for ch in chunks]

    for j in range(k):
        m = cand[0]
        for r in range(1, R):
            m = jnp.minimum(m, cand[r])
        mrow = jnp.min(m, axis=1, keepdims=True)
        knn_ref[0, j, :] = (mrow[:, 0] & 4095) + base
        cand = [jnp.where(cr == mrow, big, cr) for cr in cand]


def _compute_knn(sampled2d, coords_t, B, S, N, s_blk=256):
    kern = functools.partial(_topk_kernel, n=N, k=_K, s_blk=s_blk)
    return pl.pallas_call(
        kern,
        grid=(B, S // s_blk),
        in_specs=[
            pl.BlockSpec((s_blk, 8), lambda b, i: (b * (S // s_blk) + i, 0)),
            pl.BlockSpec((1, 8, N), lambda b, i: (b, 0, 0)),
        ],
        out_specs=pl.BlockSpec((1, _K, s_blk), lambda b, i: (b, 0, i)),
        out_shape=jax.ShapeDtypeStruct((B, _K, S), jnp.int32),
    )(sampled2d, coords_t)


def _make_sc_gather(n_rows_out, D, chunk=128):
    info = plsc.get_sparse_core_info()
    num_cores = info.num_cores
    nw = num_cores * info.num_subcores
    per_w = n_rows_out // nw
    nchunk = per_w // chunk
    mesh = plsc.VectorSubcoreMesh(core_axis_name="c", subcore_axis_name="s")

    @functools.partial(
        pl.kernel,
        mesh=mesh,
        out_type=jax.ShapeDtypeStruct((n_rows_out, D), jnp.float32),
        scratch_types=[
            pltpu.VMEM((per_w,), jnp.int32),
            pltpu.VMEM((chunk, D), jnp.float32),
            pltpu.VMEM((chunk, D), jnp.float32),
            pltpu.SemaphoreType.DMA,
            pltpu.SemaphoreType.DMA,
        ],
    )
    def gather(idx_hbm, table_hbm, out_hbm, idx_v, buf0, buf1, sem0, sem1):
        wid = lax.axis_index("s") * num_cores + lax.axis_index("c")
        base = wid * per_w
        pltpu.sync_copy(idx_hbm.at[pl.ds(base, per_w)], idx_v)

        def body(i, _):
            c0 = i * 2 * chunk
            cp0 = pltpu.async_copy(
                table_hbm.at[idx_v.at[pl.ds(c0, chunk)]], buf0, sem0)
            cp1 = pltpu.async_copy(
                table_hbm.at[idx_v.at[pl.ds(c0 + chunk, chunk)]], buf1, sem1)
            cp0.wait()
            pltpu.sync_copy(buf0, out_hbm.at[pl.ds(base + c0, chunk)])
            cp1.wait()
            pltpu.sync_copy(buf1, out_hbm.at[pl.ds(base + c0 + chunk, chunk)])
            return ()

        lax.fori_loop(0, nchunk // 2, body, (), unroll=False)

    return gather


def _gelu_exact(x):
    return 0.5 * x * (1.0 + lax.erf(x * np.float32(1.0 / np.sqrt(2.0))))


def _mlp_kernel(g_ref, samp_ref, w1c_ref, w2_ref, b2_ref, out_ref, *, k, s_blk, c_out):
    p = lax.dot(samp_ref[...], w1c_ref[...],
                precision=jax.lax.Precision.DEFAULT)
    h1 = _gelu_exact(g_ref[...] - p[None, :, :]).reshape(k * s_blk, c_out)
    h2 = _gelu_exact(
        lax.dot(h1, w2_ref[...], precision=jax.lax.Precision.DEFAULT)
        + b2_ref[...])
    acc = h2[0:s_blk]
    for j in range(1, k):
        acc = jnp.maximum(acc, h2[j * s_blk:(j + 1) * s_blk])
    out_ref[...] = acc


def _compute_mlp(G, sampled2d, W1c, W2, b2, BS, C_out, s_blk=128):
    kern = functools.partial(_mlp_kernel, k=_K, s_blk=s_blk, c_out=C_out)
    return pl.pallas_call(
        kern,
        grid=(BS // s_blk,),
        in_specs=[
            pl.BlockSpec((_K, s_blk, C_out), lambda i: (0, i, 0)),
            pl.BlockSpec((s_blk, 8), lambda i: (i, 0)),
            pl.BlockSpec((8, C_out), lambda i: (0, 0)),
            pl.BlockSpec((C_out, C_out), lambda i: (0, 0)),
            pl.BlockSpec((1, C_out), lambda i: (0, 0)),
        ],
        out_specs=pl.BlockSpec((s_blk, C_out), lambda i: (i, 0)),
        out_shape=jax.ShapeDtypeStruct((BS, C_out), jnp.float32),
    )(G, sampled2d, W1c, W2, b2)


def kernel(coords, features, W1, b1, W2, b2):
    B, N, _ = coords.shape
    C_in = features.shape[-1]
    C_out = W2.shape[-1]
    S = max(1, int(math.ceil(N * _SAMPLE_RATIO)))
    if S == N:
        idx = jnp.arange(N, dtype=jnp.int32)
    else:
        idx = jnp.round(jnp.linspace(0.0, N - 1, S)).astype(jnp.int32)

    sampled = coords[:, idx, :]

    pad = jnp.zeros((B, N, 5), jnp.float32)
    coords_p = jnp.concatenate([coords, pad], axis=-1)
    sampled_p = coords_p[:, idx, :]
    W1c = jnp.concatenate([W1[:3], jnp.zeros((5, C_out), jnp.float32)], axis=0)
    W1f = W1[3:]

    Q = _compute_q(coords_p.reshape(B * N, 8),
                   features.reshape(B * N, C_in), W1c, W1f,
                   b1.reshape(1, C_out))

    knn = _compute_knn(sampled_p.reshape(B * S, 8),
                       jnp.transpose(coords_p, (0, 2, 1)), B, S, N)
    gidx = jnp.transpose(knn, (1, 0, 2)).reshape(_K * B * S)

    G = _make_sc_gather(_K * B * S, C_out)(gidx, Q)
    G = G.reshape(_K, B * S, C_out)

    pooled = _compute_mlp(G, sampled_p.reshape(B * S, 8), W1c, W2,
                          b2.reshape(1, C_out), B * S, C_out)
    return sampled, pooled.reshape(B, S, C_out)
